# baseline (speedup 1.0000x reference)
import functools

import jax
import jax.numpy as jnp
from jax import lax
from jax.experimental import pallas as pl
from jax.experimental.pallas import tpu as pltpu
from jax.experimental.pallas import tpu_sc as plsc

_D = 128
_C = 128
_L = 16


@functools.cache
def _build_sc_kernel(n_triples: int):
    info = plsc.get_sparse_core_info()
    nc, ns = info.num_cores, info.num_subcores
    nw = nc * ns
    per_w = n_triples // nw
    assert per_w * nw == n_triples and per_w % (2 * _C) == 0
    n_half = per_w // (2 * _C)
    mesh = plsc.VectorSubcoreMesh(core_axis_name="c", subcore_axis_name="s")

    row_buf = pltpu.VMEM((_C, _D // 2), jnp.int32)

    @functools.partial(
        pl.kernel,
        mesh=mesh,
        out_type=jax.ShapeDtypeStruct((n_triples,), jnp.float32),
        compiler_params=pltpu.CompilerParams(
            needs_layout_passes=False, use_tc_tiling_on_sc=False),
        scratch_types=[
            pltpu.VMEM((per_w,), jnp.int32),
            pltpu.VMEM((per_w,), jnp.int32),
            pltpu.VMEM((per_w,), jnp.int32),
            [row_buf, row_buf, row_buf],
            [row_buf, row_buf, row_buf],
            pltpu.VMEM((2, _D // 2), jnp.int32),
            pltpu.VMEM((_L, _L), jnp.float32),
            pltpu.VMEM((_L, _L), jnp.float32),
            pltpu.VMEM((_C,), jnp.float32),
            pltpu.SemaphoreType.DMA,
            pltpu.SemaphoreType.DMA,
        ],
    )
    def sc_entail(cl_hbm, cr_hbm, d_hbm, table_hbm, out_hbm,
                  cl_ia, cr_ia, d_ia, bufs0, bufs1, bt_v, ps_d, ps_t, sc_v,
                  sem0, sem1):
        wid = lax.axis_index("s") * nc + lax.axis_index("c")
        base = wid * per_w

        pltpu.sync_copy(table_hbm.at[pl.ds(0, 2)], bt_v)

        pltpu.sync_copy(cl_hbm.at[pl.ds(base, per_w)], cl_ia)
        pltpu.sync_copy(cr_hbm.at[pl.ds(base, per_w)], cr_ia)
        pltpu.sync_copy(d_hbm.at[pl.ds(base, per_w)], d_ia)

        idx_refs = (cl_ia, cr_ia, d_ia)

        def fire(bufs, sem, ch):
            s = pl.ds(ch * _C, _C)
            for ia, buf in zip(idx_refs, bufs):
                pltpu.async_copy(table_hbm.at[ia.at[s]], buf, sem)

        def drain(bufs, sem, ch):
            s = pl.ds(ch * _C, _C)
            for ia, buf in zip(idx_refs, bufs):
                pltpu.make_async_copy(table_hbm.at[ia.at[s]], buf, sem).wait()

        lanes = lax.iota(jnp.int32, _L)
        half = jnp.bfloat16(0.5)
        n_blk = _D // (2 * _L)

        def compute(bufs, ch):
            cl_r, cr_r, d_r = bufs
            tbb = [(plsc.bitcast(bt_v[1, pl.ds(k * _L, _L)], jnp.bfloat16)
                    - plsc.bitcast(bt_v[0, pl.ds(k * _L, _L)], jnp.bfloat16))
                   * half for k in range(n_blk)]

            for g16 in range(_C // _L):
                def tri_body(i, carry2):
                    t = g16 * _L + i
                    accd = jnp.zeros((2 * _L,), jnp.bfloat16)
                    acct = jnp.zeros((2 * _L,), jnp.bfloat16)
                    for k in range(n_blk):
                        sl = pl.ds(k * _L, _L)
                        a = plsc.bitcast(cl_r[t, sl], jnp.bfloat16)
                        b = plsc.bitcast(cr_r[t, sl], jnp.bfloat16)
                        dd = plsc.bitcast(d_r[t, sl], jnp.bfloat16)
                        s = a + b
                        diff = half * s - dd
                        accd = accd + diff * diff
                        acct = acct + s * tbb[k]
                    dlo, dhi = plsc.unpack(
                        accd, format=plsc.PackFormat.INTERLEAVED)
                    tlo, thi = plsc.unpack(
                        acct, format=plsc.PackFormat.INTERLEAVED)
                    ps_d[i, :] = dlo + dhi
                    ps_t[i, :] = tlo + thi
                    return carry2

                lax.fori_loop(0, _L, tri_body, 0, unroll=4)

                accd = jnp.zeros((_L,), jnp.float32)
                acct = jnp.zeros((_L,), jnp.float32)
                for j in range(_L):
                    cols = jnp.bitwise_and(
                        jnp.full((_L,), j, jnp.int32) + lanes, _L - 1)
                    accd = accd + plsc.load_gather(ps_d, [lanes, cols])
                    acct = acct + plsc.load_gather(ps_t, [lanes, cols])

                x = accd + 1e-12
                i = plsc.bitcast(x, jnp.int32)
                i = jnp.full((_L,), 0x5F3759DF, jnp.int32) - jnp.right_shift(i, 1)
                r = plsc.bitcast(i, jnp.float32)
                for _ in range(3):
                    r = r * (1.5 - 0.5 * x * r * r)
                sc_v[pl.ds(g16 * _L, _L)] = acct - x * r

            pltpu.sync_copy(sc_v, out_hbm.at[pl.ds(base + ch * _C, _C)])

        fire(bufs0, sem0, 0)

        def pair_body(ch2, carry):
            c0 = 2 * ch2
            fire(bufs1, sem1, c0 + 1)
            drain(bufs0, sem0, c0)
            compute(bufs0, c0)

            @pl.when(ch2 + 1 < n_half)
            def _():
                fire(bufs0, sem0, c0 + 2)

            drain(bufs1, sem1, c0 + 1)
            compute(bufs1, c0 + 1)
            return carry

        lax.fori_loop(0, n_half, pair_body, 0)

    return sc_entail


def kernel(x, table):
    bs, num_axioms, ents = x.shape
    assert ents == 3
    xt = x.reshape(-1, 3).astype(jnp.int32).T
    cl, cr, d = xt[0], xt[1], xt[2]
    vocab = table.shape[0]
    tp = lax.bitcast_convert_type(
        table.astype(jnp.bfloat16).reshape(vocab, _D // 2, 2), jnp.int32)
    scores = _build_sc_kernel(bs * num_axioms)(cl, cr, d, tp)
    return scores.reshape(bs, num_axioms)


# speedup vs baseline: 1.0085x; 1.0085x over previous
import functools

import jax
import jax.numpy as jnp
from jax import lax
from jax.experimental import pallas as pl
from jax.experimental.pallas import tpu as pltpu
from jax.experimental.pallas import tpu_sc as plsc

_D = 128
_P = _D // 2
_C = 128
_L = 16
_FLUSH = 8


@functools.cache
def _build_sc_kernel(n_triples: int):
    info = plsc.get_sparse_core_info()
    nc, ns = info.num_cores, info.num_subcores
    nw = nc * ns
    per_w = n_triples // nw
    assert per_w * nw == n_triples and per_w % (2 * _C) == 0
    n_half = per_w // (2 * _C)
    mesh = plsc.VectorSubcoreMesh(core_axis_name="c", subcore_axis_name="s")

    row_buf = pltpu.VMEM((_C, _P), jnp.int32)

    @functools.partial(
        pl.kernel,
        mesh=mesh,
        out_type=jax.ShapeDtypeStruct((n_triples,), jnp.float32),
        compiler_params=pltpu.CompilerParams(
            needs_layout_passes=False, use_tc_tiling_on_sc=False),
        scratch_types=[
            pltpu.VMEM((per_w,), jnp.int32),
            pltpu.VMEM((per_w,), jnp.int32),
            pltpu.VMEM((per_w,), jnp.int32),
            [row_buf, row_buf, row_buf],
            [row_buf, row_buf, row_buf],
            pltpu.VMEM((2, _P), jnp.int32),
            pltpu.VMEM((_P,), jnp.int32),
            pltpu.VMEM((_C,), jnp.float32),
            pltpu.SemaphoreType.DMA,
            pltpu.SemaphoreType.DMA,
        ],
    )
    def sc_entail(cl_hbm, cr_hbm, d_hbm, table_hbm, out_hbm,
                  cl_ia, cr_ia, d_ia, bufs0, bufs1, bt_v, tbp_v, sc_v,
                  sem0, sem1):
        wid = lax.axis_index("s") * nc + lax.axis_index("c")
        base = wid * per_w

        pltpu.sync_copy(table_hbm.at[pl.ds(0, 2)], bt_v)
        for w in range(_P // _L):
            sl = pl.ds(w * _L, _L)
            bot = plsc.bitcast(bt_v[0, sl], jnp.bfloat16)
            top = plsc.bitcast(bt_v[1, sl], jnp.bfloat16)
            tbh = (top - bot) * jnp.bfloat16(0.5)
            tbp_v[sl] = plsc.bitcast(tbh, jnp.int32)

        pltpu.sync_copy(cl_hbm.at[pl.ds(base, per_w)], cl_ia)
        pltpu.sync_copy(cr_hbm.at[pl.ds(base, per_w)], cr_ia)
        pltpu.sync_copy(d_hbm.at[pl.ds(base, per_w)], d_ia)

        idx_refs = (cl_ia, cr_ia, d_ia)

        def fire(bufs, sem, ch):
            s = pl.ds(ch * _C, _C)
            for ia, buf in zip(idx_refs, bufs):
                pltpu.async_copy(table_hbm.at[ia.at[s]], buf, sem)

        def drain(bufs, sem, ch):
            s = pl.ds(ch * _C, _C)
            for ia, buf in zip(idx_refs, bufs):
                pltpu.make_async_copy(table_hbm.at[ia.at[s]], buf, sem).wait()

        lanes = lax.iota(jnp.int32, _L)
        half = jnp.bfloat16(0.5)
        n_g = _C // _L

        rows_list = [jnp.full((_L,), g * _L, jnp.int32) + lanes
                     for g in range(n_g)]

        def compute(bufs, ch):
            cl_r, cr_r, d_r = bufs

            def blk_body(co, carry2):
                accd_bf = [jnp.zeros((2 * _L,), jnp.bfloat16)
                           for _ in range(n_g)]
                acct_bf = [jnp.zeros((2 * _L,), jnp.bfloat16)
                           for _ in range(n_g)]
                for j in range(_FLUSH):
                    cp = co * _FLUSH + j
                    cols = jnp.bitwise_and(
                        jnp.full((_L,), cp, jnp.int32) + lanes, _P - 1)
                    tbv = plsc.bitcast(
                        plsc.load_gather(tbp_v, [cols]), jnp.bfloat16)
                    for g in range(n_g):
                        a = plsc.bitcast(
                            plsc.load_gather(cl_r, [rows_list[g], cols]),
                            jnp.bfloat16)
                        b = plsc.bitcast(
                            plsc.load_gather(cr_r, [rows_list[g], cols]),
                            jnp.bfloat16)
                        dd = plsc.bitcast(
                            plsc.load_gather(d_r, [rows_list[g], cols]),
                            jnp.bfloat16)
                        s = a + b
                        diff = half * s - dd
                        accd_bf[g] = accd_bf[g] + diff * diff
                        acct_bf[g] = acct_bf[g] + s * tbv
                new = []
                for g in range(n_g):
                    dlo, dhi = plsc.unpack(
                        accd_bf[g], format=plsc.PackFormat.INTERLEAVED)
                    tlo, thi = plsc.unpack(
                        acct_bf[g], format=plsc.PackFormat.INTERLEAVED)
                    new.append(carry2[2 * g] + (dlo + dhi))
                    new.append(carry2[2 * g + 1] + (tlo + thi))
                return tuple(new)

            accs = lax.fori_loop(
                0, _P // _FLUSH, blk_body,
                tuple(jnp.zeros((_L,), jnp.float32) for _ in range(2 * n_g)))

            for g in range(n_g):
                accd, acct = accs[2 * g], accs[2 * g + 1]
                x = accd + 1e-12
                i = plsc.bitcast(x, jnp.int32)
                i = jnp.full((_L,), 0x5F3759DF, jnp.int32) - jnp.right_shift(i, 1)
                r = plsc.bitcast(i, jnp.float32)
                for _ in range(3):
                    r = r * (1.5 - 0.5 * x * r * r)
                sc_v[pl.ds(g * _L, _L)] = acct - x * r

            pltpu.sync_copy(sc_v, out_hbm.at[pl.ds(base + ch * _C, _C)])

        fire(bufs0, sem0, 0)

        def pair_body(ch2, carry):
            c0 = 2 * ch2
            fire(bufs1, sem1, c0 + 1)
            drain(bufs0, sem0, c0)
            compute(bufs0, c0)

            @pl.when(ch2 + 1 < n_half)
            def _():
                fire(bufs0, sem0, c0 + 2)

            drain(bufs1, sem1, c0 + 1)
            compute(bufs1, c0 + 1)
            return carry

        lax.fori_loop(0, n_half, pair_body, 0)

    return sc_entail


def kernel(x, table):
    bs, num_axioms, ents = x.shape
    assert ents == 3
    xt = x.reshape(-1, 3).astype(jnp.int32).T
    cl, cr, d = xt[0], xt[1], xt[2]
    vocab = table.shape[0]
    tp = lax.bitcast_convert_type(
        table.astype(jnp.bfloat16).reshape(vocab, _P, 2), jnp.int32)
    scores = _build_sc_kernel(bs * num_axioms)(cl, cr, d, tp)
    return scores.reshape(bs, num_axioms)


# speedup vs baseline: 1.9418x; 1.9255x over previous
import functools

import jax
import jax.numpy as jnp
from jax import lax
from jax.experimental import pallas as pl
from jax.experimental.pallas import tpu as pltpu
from jax.experimental.pallas import tpu_sc as plsc

_D = 128
_P = _D // 2
_C = 128
_L = 16
_FLUSH = 8


@functools.cache
def _build_sc_kernel(n_triples: int):
    info = plsc.get_sparse_core_info()
    nc, ns = info.num_cores, info.num_subcores
    nw = nc * ns
    per_w = n_triples // nw
    assert per_w * nw == n_triples and per_w % (2 * _C) == 0
    n_half = per_w // (2 * _C)
    mesh = plsc.VectorSubcoreMesh(core_axis_name="c", subcore_axis_name="s")

    row_buf = pltpu.VMEM((_C, _P), jnp.int32)

    @functools.partial(
        pl.kernel,
        mesh=mesh,
        out_type=jax.ShapeDtypeStruct((n_triples,), jnp.float32),
        compiler_params=pltpu.CompilerParams(
            needs_layout_passes=False, use_tc_tiling_on_sc=False),
        scratch_types=[
            pltpu.VMEM((per_w,), jnp.int32),
            pltpu.VMEM((per_w,), jnp.int32),
            pltpu.VMEM((per_w,), jnp.int32),
            [row_buf, row_buf, row_buf],
            [row_buf, row_buf, row_buf],
            pltpu.VMEM((2, _P), jnp.int32),
            pltpu.VMEM((_P,), jnp.int32),
            pltpu.VMEM((_C,), jnp.float32),
            pltpu.SemaphoreType.DMA,
            pltpu.SemaphoreType.DMA,
        ],
    )
    def sc_entail(cl_hbm, cr_hbm, d_hbm, table_hbm, out_hbm,
                  cl_ia, cr_ia, d_ia, bufs0, bufs1, bt_v, tbp_v, sc_v,
                  sem0, sem1):
        wid = lax.axis_index("s") * nc + lax.axis_index("c")
        base = wid * per_w

        pltpu.sync_copy(table_hbm.at[pl.ds(0, 2)], bt_v)
        for w in range(_P // _L):
            sl = pl.ds(w * _L, _L)
            bot = plsc.bitcast(bt_v[0, sl], jnp.bfloat16)
            top = plsc.bitcast(bt_v[1, sl], jnp.bfloat16)
            tbh = (top - bot) * jnp.bfloat16(0.5)
            tbp_v[sl] = plsc.bitcast(tbh, jnp.int32)

        pltpu.sync_copy(cl_hbm.at[pl.ds(base, per_w)], cl_ia)
        pltpu.sync_copy(cr_hbm.at[pl.ds(base, per_w)], cr_ia)
        pltpu.sync_copy(d_hbm.at[pl.ds(base, per_w)], d_ia)

        idx_refs = (cl_ia, cr_ia, d_ia)

        def fire(bufs, sem, ch):
            s = pl.ds(ch * _C, _C)
            for ia, buf in zip(idx_refs, bufs):
                pltpu.async_copy(table_hbm.at[ia.at[s]], buf, sem)

        def drain(bufs, sem, ch):
            s = pl.ds(ch * _C, _C)
            for ia, buf in zip(idx_refs, bufs):
                pltpu.make_async_copy(table_hbm.at[ia.at[s]], buf, sem).wait()

        lanes = lax.iota(jnp.int32, _L)
        half = jnp.bfloat16(0.5)
        n_g = _C // _L

        rows_list = [jnp.full((_L,), g * _L, jnp.int32) + lanes
                     for g in range(n_g)]

        def compute(bufs, ch):
            cl_r, cr_r, d_r = bufs

            def blk_body(co, carry2):
                accd_bf = [jnp.zeros((2 * _L,), jnp.bfloat16)
                           for _ in range(n_g)]
                acct_bf = [jnp.zeros((2 * _L,), jnp.bfloat16)
                           for _ in range(n_g)]
                for j in range(_FLUSH):
                    cp = co * _FLUSH + j
                    cols = jnp.bitwise_and(
                        jnp.full((_L,), cp, jnp.int32) + lanes, _P - 1)
                    tbv = plsc.bitcast(
                        plsc.load_gather(tbp_v, [cols]), jnp.bfloat16)
                    for g in range(n_g):
                        a = plsc.bitcast(
                            plsc.load_gather(cl_r, [rows_list[g], cols]),
                            jnp.bfloat16)
                        b = plsc.bitcast(
                            plsc.load_gather(cr_r, [rows_list[g], cols]),
                            jnp.bfloat16)
                        dd = plsc.bitcast(
                            plsc.load_gather(d_r, [rows_list[g], cols]),
                            jnp.bfloat16)
                        s = a + b
                        diff = half * s - dd
                        accd_bf[g] = accd_bf[g] + diff * diff
                        acct_bf[g] = acct_bf[g] + s * tbv
                new = []
                for g in range(n_g):
                    dlo, dhi = plsc.unpack(
                        accd_bf[g], format=plsc.PackFormat.INTERLEAVED)
                    tlo, thi = plsc.unpack(
                        acct_bf[g], format=plsc.PackFormat.INTERLEAVED)
                    new.append(carry2[2 * g] + (dlo + dhi))
                    new.append(carry2[2 * g + 1] + (tlo + thi))
                return tuple(new)

            accs = lax.fori_loop(
                0, _P // _FLUSH, blk_body,
                tuple(jnp.zeros((_L,), jnp.float32) for _ in range(2 * n_g)))

            for g in range(n_g):
                accd, acct = accs[2 * g], accs[2 * g + 1]
                x = accd + 1e-12
                i = plsc.bitcast(x, jnp.int32)
                i = jnp.full((_L,), 0x5F3759DF, jnp.int32) - jnp.right_shift(i, 1)
                r = plsc.bitcast(i, jnp.float32)
                for _ in range(3):
                    r = r * (1.5 - 0.5 * x * r * r)
                sc_v[pl.ds(g * _L, _L)] = acct - x * r

            pltpu.sync_copy(sc_v, out_hbm.at[pl.ds(base + ch * _C, _C)])

        fire(bufs0, sem0, 0)

        def pair_body(ch2, carry):
            c0 = 2 * ch2
            fire(bufs1, sem1, c0 + 1)
            drain(bufs0, sem0, c0)
            compute(bufs0, c0)

            @pl.when(ch2 + 1 < n_half)
            def _():
                fire(bufs0, sem0, c0 + 2)

            drain(bufs1, sem1, c0 + 1)
            compute(bufs1, c0 + 1)
            return carry

        lax.fori_loop(0, n_half, pair_body, 0)

    return sc_entail


def kernel(x, table):
    bs, num_axioms, ents = x.shape
    assert ents == 3
    xt = x.reshape(-1, 3).astype(jnp.int32).T
    cl, cr, d = xt[0], xt[1], xt[2]
    bits = lax.bitcast_convert_type(table, jnp.int32)
    b16 = jnp.right_shift(
        bits + 0x7FFF + jnp.bitwise_and(jnp.right_shift(bits, 16), 1), 16)
    b16 = jnp.bitwise_and(b16, 0xFFFF)
    tp = jnp.bitwise_or(b16[:, :_P], jnp.left_shift(b16[:, _P:], 16))
    scores = _build_sc_kernel(bs * num_axioms)(cl, cr, d, tp)
    return scores.reshape(bs, num_axioms)


# speedup vs baseline: 4.1720x; 2.1485x over previous
import functools

import jax
import jax.numpy as jnp
from jax import lax
from jax.experimental import pallas as pl
from jax.experimental.pallas import tpu as pltpu
from jax.experimental.pallas import tpu_sc as plsc

_D = 128
_C = 128
_L = 16


@functools.cache
def _build_sc_kernel(n_triples: int):
    info = plsc.get_sparse_core_info()
    nc, ns = info.num_cores, info.num_subcores
    nw = nc * ns
    per_w = n_triples // nw
    assert per_w * nw == n_triples and per_w % (2 * _C) == 0
    n_half = per_w // (2 * _C)
    mesh = plsc.VectorSubcoreMesh(core_axis_name="c", subcore_axis_name="s")

    row_buf = pltpu.VMEM((_C, _D), jnp.float32)

    @functools.partial(
        pl.kernel,
        mesh=mesh,
        out_type=jax.ShapeDtypeStruct((n_triples,), jnp.float32),
        compiler_params=pltpu.CompilerParams(needs_layout_passes=False),
        scratch_types=[
            pltpu.VMEM((per_w,), jnp.int32),
            pltpu.VMEM((per_w,), jnp.int32),
            pltpu.VMEM((per_w,), jnp.int32),
            [row_buf, row_buf, row_buf],
            [row_buf, row_buf, row_buf],
            pltpu.VMEM((2, _D), jnp.float32),
            pltpu.VMEM((_D,), jnp.float32),
            pltpu.VMEM((_C,), jnp.float32),
            pltpu.SemaphoreType.DMA,
            pltpu.SemaphoreType.DMA,
        ],
    )
    def sc_entail(cl_hbm, cr_hbm, d_hbm, table_hbm, out_hbm,
                  cl_ia, cr_ia, d_ia, bufs0, bufs1, bt_v, tbh_v, sc_v,
                  sem0, sem1):
        wid = lax.axis_index("s") * nc + lax.axis_index("c")
        base = wid * per_w

        pltpu.sync_copy(table_hbm.at[pl.ds(0, 2)], bt_v)
        for g in range(_D // _L):
            sl = pl.ds(g * _L, _L)
            tbh_v[sl] = 0.5 * (bt_v[1, sl] - bt_v[0, sl])

        pltpu.sync_copy(cl_hbm.at[pl.ds(base, per_w)], cl_ia)
        pltpu.sync_copy(cr_hbm.at[pl.ds(base, per_w)], cr_ia)
        pltpu.sync_copy(d_hbm.at[pl.ds(base, per_w)], d_ia)

        idx_refs = (cl_ia, cr_ia, d_ia)

        def fire(bufs, sem, ch):
            s = pl.ds(ch * _C, _C)
            for ia, buf in zip(idx_refs, bufs):
                pltpu.async_copy(table_hbm.at[ia.at[s]], buf, sem)

        def drain(bufs, sem, ch):
            s = pl.ds(ch * _C, _C)
            for ia, buf in zip(idx_refs, bufs):
                pltpu.make_async_copy(table_hbm.at[ia.at[s]], buf, sem).wait()

        def compute(bufs, ch):
            cl_r, cr_r, d_r = bufs
            lanes = lax.iota(jnp.int32, _L)
            n_g = _C // _L
            rows_list = [jnp.full((_L,), g * _L, jnp.int32) + lanes
                         for g in range(n_g)]

            def col_body(c, carry2):
                cols = jnp.bitwise_and(
                    jnp.full((_L,), c, jnp.int32) + lanes, _D - 1)
                tb = plsc.load_gather(tbh_v, [cols])
                new = []
                for g in range(n_g):
                    a = plsc.load_gather(cl_r, [rows_list[g], cols])
                    b = plsc.load_gather(cr_r, [rows_list[g], cols])
                    dd = plsc.load_gather(d_r, [rows_list[g], cols])
                    s = a + b
                    diff = 0.5 * s - dd
                    new.append(carry2[2 * g] + diff * diff)
                    new.append(carry2[2 * g + 1] + s * tb)
                return tuple(new)

            accs = lax.fori_loop(
                0, _D, col_body,
                tuple(jnp.zeros((_L,), jnp.float32) for _ in range(2 * n_g)),
                unroll=4)

            for g in range(n_g):
                accd, acct = accs[2 * g], accs[2 * g + 1]
                x = accd + 1e-12
                i = plsc.bitcast(x, jnp.int32)
                i = jnp.full((_L,), 0x5F3759DF, jnp.int32) - jnp.right_shift(i, 1)
                r = plsc.bitcast(i, jnp.float32)
                for _ in range(3):
                    r = r * (1.5 - 0.5 * x * r * r)
                sc_v[pl.ds(g * _L, _L)] = acct - x * r

            pltpu.sync_copy(sc_v, out_hbm.at[pl.ds(base + ch * _C, _C)])

        fire(bufs0, sem0, 0)

        def pair_body(ch2, carry):
            c0 = 2 * ch2
            fire(bufs1, sem1, c0 + 1)
            drain(bufs0, sem0, c0)
            compute(bufs0, c0)

            @pl.when(ch2 + 1 < n_half)
            def _():
                fire(bufs0, sem0, c0 + 2)

            drain(bufs1, sem1, c0 + 1)
            compute(bufs1, c0 + 1)
            return carry

        lax.fori_loop(0, n_half, pair_body, 0)

    return sc_entail


def kernel(x, table):
    bs, num_axioms, ents = x.shape
    assert ents == 3
    xt = x.reshape(-1, 3).astype(jnp.int32).T
    cl, cr, d = xt[0], xt[1], xt[2]
    scores = _build_sc_kernel(bs * num_axioms)(cl, cr, d, table)
    return scores.reshape(bs, num_axioms)
